# baseline (speedup 1.0000x reference)
import jax
import jax.numpy as jnp
from jax import lax
from jax.experimental import pallas as pl
from jax.experimental.pallas import tpu as pltpu

_STEPS = 16
_NBUF = 4
_D = 1000
_K = 64
_NU = 16384
_NV = 4096
_BU = _NU // _STEPS
_BV = _NV // _STEPS


def _fused_body(eut_ref, evt_ref, ut_hbm, vt_hbm, pt_hbm, qt_hbm,
                ub, vb, pb, qb, usem, vsem, psem, qsem):

    def fetch(i):
        s = lax.rem(i, _NBUF)
        pltpu.make_async_copy(
            ut_hbm.at[:, pl.ds(i * _BU, _BU)], ub.at[s], usem.at[s]).start()
        pltpu.make_async_copy(
            vt_hbm.at[:, pl.ds(i * _BV, _BV)], vb.at[s], vsem.at[s]).start()

    for i in range(_NBUF - 1):
        fetch(jnp.int32(i))

    def step(i, carry):
        s = lax.rem(i, _NBUF)
        os = lax.rem(i, 2)

        @pl.when(i >= 2)
        def _wait_out():
            j = i - 2
            pltpu.make_async_copy(
                pb.at[os], pt_hbm.at[:, pl.ds(j * _BU, _BU)], psem.at[os]).wait()
            pltpu.make_async_copy(
                qb.at[os], qt_hbm.at[:, pl.ds(j * _BV, _BV)], qsem.at[os]).wait()

        @pl.when(i + _NBUF - 1 < _STEPS)
        def _prefetch():
            fetch(i + _NBUF - 1)

        pltpu.make_async_copy(
            ut_hbm.at[:, pl.ds(i * _BU, _BU)], ub.at[s], usem.at[s]).wait()
        pltpu.make_async_copy(
            vt_hbm.at[:, pl.ds(i * _BV, _BV)], vb.at[s], vsem.at[s]).wait()

        pb[os] = jnp.dot(eut_ref[...], ub[s],
                         preferred_element_type=jnp.float32)
        qb[os] = jnp.dot(evt_ref[...], vb[s],
                         preferred_element_type=jnp.float32)

        pltpu.make_async_copy(
            pb.at[os], pt_hbm.at[:, pl.ds(i * _BU, _BU)], psem.at[os]).start()
        pltpu.make_async_copy(
            qb.at[os], qt_hbm.at[:, pl.ds(i * _BV, _BV)], qsem.at[os]).start()
        return carry

    lax.fori_loop(0, _STEPS, step, 0)

    for j in (_STEPS - 2, _STEPS - 1):
        os = j % 2
        pltpu.make_async_copy(
            pb.at[os], pt_hbm.at[:, pl.ds(j * _BU, _BU)], psem.at[os]).wait()
        pltpu.make_async_copy(
            qb.at[os], qt_hbm.at[:, pl.ds(j * _BV, _BV)], qsem.at[os]).wait()


def kernel(U, V, Eu, Ev):
    Ut, Vt, EuT, EvT = U.T, V.T, Eu.T, Ev.T
    Pt, Qt = pl.pallas_call(
        _fused_body,
        in_specs=[
            pl.BlockSpec((_K, _D), lambda: (0, 0)),
            pl.BlockSpec((_K, _D), lambda: (0, 0)),
            pl.BlockSpec(memory_space=pl.ANY),
            pl.BlockSpec(memory_space=pl.ANY),
        ],
        out_specs=[
            pl.BlockSpec(memory_space=pl.ANY),
            pl.BlockSpec(memory_space=pl.ANY),
        ],
        out_shape=[
            jax.ShapeDtypeStruct((_K, _NU), jnp.float32),
            jax.ShapeDtypeStruct((_K, _NV), jnp.float32),
        ],
        scratch_shapes=[
            pltpu.VMEM((_NBUF, _D, _BU), jnp.float32),
            pltpu.VMEM((_NBUF, _D, _BV), jnp.float32),
            pltpu.VMEM((2, _K, _BU), jnp.float32),
            pltpu.VMEM((2, _K, _BV), jnp.float32),
            pltpu.SemaphoreType.DMA((_NBUF,)),
            pltpu.SemaphoreType.DMA((_NBUF,)),
            pltpu.SemaphoreType.DMA((2,)),
            pltpu.SemaphoreType.DMA((2,)),
        ],
        compiler_params=pltpu.CompilerParams(
            dimension_semantics=(),
            disable_bounds_checks=True,
        ),
    )(EuT, EvT, Ut, Vt)
    return (Pt.T, Qt.T)


# speedup vs baseline: 1.0119x; 1.0119x over previous
import jax
import jax.numpy as jnp
from jax import lax
from jax.experimental import pallas as pl
from jax.experimental.pallas import tpu as pltpu

_STEPS = 32
_NBUF = 8
_D = 1000
_K = 64
_NU = 16384
_NV = 4096
_BU = _NU // _STEPS
_BV = _NV // _STEPS


def _fused_body(eut_ref, evt_ref, ut_hbm, vt_hbm, pt_hbm, qt_hbm,
                ub, vb, pb, qb, usem, vsem, psem, qsem):

    def fetch(i):
        s = lax.rem(i, _NBUF)
        pltpu.make_async_copy(
            ut_hbm.at[:, pl.ds(i * _BU, _BU)], ub.at[s], usem.at[s]).start()
        pltpu.make_async_copy(
            vt_hbm.at[:, pl.ds(i * _BV, _BV)], vb.at[s], vsem.at[s]).start()

    for i in range(_NBUF - 1):
        fetch(jnp.int32(i))

    def step(i, carry):
        s = lax.rem(i, _NBUF)
        os = lax.rem(i, 2)

        @pl.when(i >= 2)
        def _wait_out():
            j = i - 2
            pltpu.make_async_copy(
                pb.at[os], pt_hbm.at[:, pl.ds(j * _BU, _BU)], psem.at[os]).wait()
            pltpu.make_async_copy(
                qb.at[os], qt_hbm.at[:, pl.ds(j * _BV, _BV)], qsem.at[os]).wait()

        @pl.when(i + _NBUF - 1 < _STEPS)
        def _prefetch():
            fetch(i + _NBUF - 1)

        pltpu.make_async_copy(
            ut_hbm.at[:, pl.ds(i * _BU, _BU)], ub.at[s], usem.at[s]).wait()
        pltpu.make_async_copy(
            vt_hbm.at[:, pl.ds(i * _BV, _BV)], vb.at[s], vsem.at[s]).wait()

        pb[os] = jnp.dot(eut_ref[...], ub[s],
                         preferred_element_type=jnp.float32)
        qb[os] = jnp.dot(evt_ref[...], vb[s],
                         preferred_element_type=jnp.float32)

        pltpu.make_async_copy(
            pb.at[os], pt_hbm.at[:, pl.ds(i * _BU, _BU)], psem.at[os]).start()
        pltpu.make_async_copy(
            qb.at[os], qt_hbm.at[:, pl.ds(i * _BV, _BV)], qsem.at[os]).start()
        return carry

    lax.fori_loop(0, _STEPS, step, 0)

    for j in (_STEPS - 2, _STEPS - 1):
        os = j % 2
        pltpu.make_async_copy(
            pb.at[os], pt_hbm.at[:, pl.ds(j * _BU, _BU)], psem.at[os]).wait()
        pltpu.make_async_copy(
            qb.at[os], qt_hbm.at[:, pl.ds(j * _BV, _BV)], qsem.at[os]).wait()


def kernel(U, V, Eu, Ev):
    Ut, Vt, EuT, EvT = U.T, V.T, Eu.T, Ev.T
    Pt, Qt = pl.pallas_call(
        _fused_body,
        in_specs=[
            pl.BlockSpec((_K, _D), lambda: (0, 0)),
            pl.BlockSpec((_K, _D), lambda: (0, 0)),
            pl.BlockSpec(memory_space=pl.ANY),
            pl.BlockSpec(memory_space=pl.ANY),
        ],
        out_specs=[
            pl.BlockSpec(memory_space=pl.ANY),
            pl.BlockSpec(memory_space=pl.ANY),
        ],
        out_shape=[
            jax.ShapeDtypeStruct((_K, _NU), jnp.float32),
            jax.ShapeDtypeStruct((_K, _NV), jnp.float32),
        ],
        scratch_shapes=[
            pltpu.VMEM((_NBUF, _D, _BU), jnp.float32),
            pltpu.VMEM((_NBUF, _D, _BV), jnp.float32),
            pltpu.VMEM((2, _K, _BU), jnp.float32),
            pltpu.VMEM((2, _K, _BV), jnp.float32),
            pltpu.SemaphoreType.DMA((_NBUF,)),
            pltpu.SemaphoreType.DMA((_NBUF,)),
            pltpu.SemaphoreType.DMA((2,)),
            pltpu.SemaphoreType.DMA((2,)),
        ],
        compiler_params=pltpu.CompilerParams(
            dimension_semantics=(),
            disable_bounds_checks=True,
        ),
    )(EuT, EvT, Ut, Vt)
    return (Pt.T, Qt.T)


# speedup vs baseline: 1.0123x; 1.0004x over previous
import jax
import jax.numpy as jnp
from jax.experimental import pallas as pl
from jax.experimental.pallas import tpu as pltpu


def _fused_kernel(eut_ref, evt_ref, ut_ref, vt_ref, pt_ref, qt_ref):
    pt_ref[...] = jnp.dot(eut_ref[...], ut_ref[...],
                          preferred_element_type=jnp.float32)
    qt_ref[...] = jnp.dot(evt_ref[...], vt_ref[...],
                          preferred_element_type=jnp.float32)


def kernel(U, V, Eu, Ev):
    Ut, Vt, EuT, EvT = U.T, V.T, Eu.T, Ev.T
    d, nu = Ut.shape
    _, nv = Vt.shape
    k = EuT.shape[0]
    steps = 16
    bu = nu // steps
    bv = nv // steps
    Pt, Qt = pl.pallas_call(
        _fused_kernel,
        grid=(steps,),
        in_specs=[
            pl.BlockSpec((k, d), lambda i: (0, 0)),
            pl.BlockSpec((k, d), lambda i: (0, 0)),
            pl.BlockSpec((d, bu), lambda i: (0, i)),
            pl.BlockSpec((d, bv), lambda i: (0, i)),
        ],
        out_specs=[
            pl.BlockSpec((k, bu), lambda i: (0, i)),
            pl.BlockSpec((k, bv), lambda i: (0, i)),
        ],
        out_shape=[
            jax.ShapeDtypeStruct((k, nu), jnp.float32),
            jax.ShapeDtypeStruct((k, nv), jnp.float32),
        ],
        compiler_params=pltpu.CompilerParams(
            dimension_semantics=(pltpu.ARBITRARY,),
            disable_bounds_checks=True,
        ),
    )(EuT, EvT, Ut, Vt)
    return (Pt.T, Qt.T)
